# baseline (speedup 1.0000x reference)
import jax
import jax.numpy as jnp
from jax import lax
from jax.experimental import pallas as pl
from jax.experimental.pallas import tpu as pltpu
from jax.experimental.pallas import tpu_sc as plsc

_SEQ = 8192
_DIM = 768
_BATCH = 4
_NW = 32
_ROWS_PER_W = _SEQ // _NW
_SUB = 16
_BLK = _SUB * _DIM
_NSB = _ROWS_PER_W // _SUB


def _sc_body(in_hbm, pos_hbm, out_hbm, in_bufs, pos_bufs, in_sems, pos_sems, out_sems):
    c = lax.axis_index("c")
    s = lax.axis_index("s")
    wid = s * 2 + c
    base_row = wid * _ROWS_PER_W

    def in_off(sb, b):
        return (b * _SEQ + base_row + sb * _SUB) * _DIM

    def pos_off(sb):
        return (base_row + sb * _SUB) * _DIM

    def start_pos(sb, p):
        pltpu.async_copy(
            pos_hbm.at[pl.ds(pos_off(sb), _BLK)],
            pos_bufs.at[pl.ds(p * _BLK, _BLK)],
            pos_sems.at[p],
        )

    def wait_pos(sb, p):
        pltpu.make_async_copy(
            pos_hbm.at[pl.ds(pos_off(sb), _BLK)],
            pos_bufs.at[pl.ds(p * _BLK, _BLK)],
            pos_sems.at[p],
        ).wait()

    def start_in(sb, p, b):
        g = p * _BATCH + b
        pltpu.async_copy(
            in_hbm.at[pl.ds(in_off(sb, b), _BLK)],
            in_bufs.at[pl.ds(g * _BLK, _BLK)],
            in_sems.at[g],
        )

    def wait_in(sb, p, b):
        g = p * _BATCH + b
        pltpu.make_async_copy(
            in_hbm.at[pl.ds(in_off(sb, b), _BLK)],
            in_bufs.at[pl.ds(g * _BLK, _BLK)],
            in_sems.at[g],
        ).wait()

    def start_out(sb, p, b):
        g = p * _BATCH + b
        pltpu.async_copy(
            in_bufs.at[pl.ds(g * _BLK, _BLK)],
            out_hbm.at[pl.ds(in_off(sb, b), _BLK)],
            out_sems.at[g],
        )

    def wait_out(sb, p, b):
        g = p * _BATCH + b
        pltpu.make_async_copy(
            in_bufs.at[pl.ds(g * _BLK, _BLK)],
            out_hbm.at[pl.ds(in_off(sb, b), _BLK)],
            out_sems.at[g],
        ).wait()

    for p in range(2):
        start_pos(p, p)
        for b in range(_BATCH):
            start_in(p, p, b)

    def super_iter(k, carry):
        for p in range(2):
            sb = 2 * k + p
            wait_pos(sb, p)
            for b in range(_BATCH):
                g = p * _BATCH + b
                wait_in(sb, p, b)

                def add_iter(i, _, g=g, p=p):
                    dst = pl.ds(g * _BLK + i * 16, 16)
                    src = pl.ds(p * _BLK + i * 16, 16)
                    in_bufs[dst] = in_bufs[dst] + pos_bufs[src]
                    return 0

                lax.fori_loop(0, _BLK // 16, add_iter, 0, unroll=8)
                start_out(sb, p, b)

            @pl.when(2 * k + p + 2 < _NSB)
            def _prefetch(p=p, sb=sb):
                sb2 = sb + 2
                for b in range(_BATCH):
                    wait_out(sb, p, b)
                    start_in(sb2, p, b)
                start_pos(sb2, p)

        return carry

    lax.fori_loop(0, _NSB // 2, super_iter, 0)

    for p in range(2):
        sb = _NSB - 2 + p
        for b in range(_BATCH):
            wait_out(sb, p, b)


def kernel(inputs, pos_table):
    in_flat = inputs.reshape(-1)
    pos_flat = pos_table.reshape(-1)
    mesh = plsc.VectorSubcoreMesh(core_axis_name="c", subcore_axis_name="s")
    out = pl.kernel(
        _sc_body,
        mesh=mesh,
        out_type=jax.ShapeDtypeStruct((_BATCH * _SEQ * _DIM,), jnp.float32),
        scratch_types=[
            pltpu.VMEM((2 * _BATCH * _BLK,), jnp.float32),
            pltpu.VMEM((2 * _BLK,), jnp.float32),
            pltpu.SemaphoreType.DMA((2 * _BATCH,)),
            pltpu.SemaphoreType.DMA((2,)),
            pltpu.SemaphoreType.DMA((2 * _BATCH,)),
        ],
    )(in_flat, pos_flat)
    return out.reshape(inputs.shape)


# speedup vs baseline: 4.9665x; 4.9665x over previous
import jax
import jax.numpy as jnp
from jax.experimental import pallas as pl

_SEQ_BLOCK = 1024


def _add_kernel(in_ref, pos_ref, out_ref):
    out_ref[...] = in_ref[...] + pos_ref[...][None, :, :]


def kernel(inputs, pos_table):
    batch, seq, dim = inputs.shape
    grid = (seq // _SEQ_BLOCK,)
    return pl.pallas_call(
        _add_kernel,
        grid=grid,
        in_specs=[
            pl.BlockSpec((batch, _SEQ_BLOCK, dim), lambda i: (0, i, 0)),
            pl.BlockSpec((_SEQ_BLOCK, dim), lambda i: (i, 0)),
        ],
        out_specs=pl.BlockSpec((batch, _SEQ_BLOCK, dim), lambda i: (0, i, 0)),
        out_shape=jax.ShapeDtypeStruct(inputs.shape, inputs.dtype),
    )(inputs, pos_table)
